# Initial kernel scaffold; baseline (speedup 1.0000x reference)
import jax
import jax.numpy as jnp
from jax.experimental import pallas as pl


def kernel(coord, hier_indices, features):
    raise NotImplementedError("write your pallas kernel here")


# speedup vs baseline: 10.9183x; 10.9183x over previous
import jax
import jax.numpy as jnp
from jax import lax
from jax.experimental import pallas as pl
from jax.experimental.pallas import tpu as pltpu
from jax.experimental.pallas import tpu_sc as plsc

_N = 500000
_D = 8
_R = 1000001
_LEVELS = 3
_B = 160
_NBLK = _N // _B
_IDXW = 128
_IDXROWS = _B * 8 // _IDXW
_NC = 2
_NS = 16
_NW = _NC * _NS


def _octree_body(coord_hbm, hier_hbm, feat_hbm, out_hbm,
                 idx_v, g0, g1, g2, cx_v, cy_v, cz_v, out_v, sem):
    g_refs = (g0, g1, g2)
    wid = lax.axis_index("s") * _NC + lax.axis_index("c")
    nblk_w = (_NBLK - wid + _NW - 1) // _NW
    lanes = lax.iota(jnp.int32, 16)

    def block_body(k, _):
        blk = wid + k * _NW
        base = blk * _B

        for i in range(_LEVELS):
            pltpu.sync_copy(
                hier_hbm.at[pl.ds(i * (_NBLK * _IDXROWS) + blk * _IDXROWS,
                                  _IDXROWS), :],
                idx_v.at[i])
        cps = []
        for i in range(_LEVELS):
            for j in range(_IDXROWS):
                cps.append(pltpu.async_copy(
                    feat_hbm.at[idx_v.at[i, j]],
                    g_refs[i].at[pl.ds(j * _IDXW, _IDXW), :],
                    sem))
        pltpu.sync_copy(coord_hbm.at[pl.ds(base, _B)], cx_v)
        pltpu.sync_copy(coord_hbm.at[pl.ds(_N + base, _B)], cy_v)
        pltpu.sync_copy(coord_hbm.at[pl.ds(2 * _N + base, _B)], cz_v)
        for cp in cps:
            cp.wait()

        def group_body(gi, _):
            p0 = gi * 16
            plocal = p0 + lanes
            rb = plocal * 8
            cx = cx_v[pl.ds(p0, 16)]
            cy = cy_v[pl.ds(p0, 16)]
            cz = cz_v[pl.ds(p0, 16)]
            acc = [jnp.zeros((16,), jnp.float32) for _ in range(_D)]
            for i in range(_LEVELS):
                s = jnp.float32(2.0 ** (11 - i))
                ts = []
                for cc in (cx, cy, cz):
                    coords = cc * s + s
                    d = coords - lax.convert_element_type(
                        lax.convert_element_type(coords, jnp.int32),
                        jnp.float32)
                    ts.append(d * d * (3.0 - 2.0 * d))
                tx, ty, tz = ts
                ax = (1.0 - tx, tx)
                ay = (1.0 - ty, ty)
                az = (1.0 - tz, tz)
                axy = tuple(ax[b2] * ay[b1]
                            for b2 in range(2) for b1 in range(2))
                gref = g_refs[i]
                for c in range(8):
                    w = axy[c >> 1] * az[c & 1]
                    rows = rb + c
                    for dd in range(_D):
                        col = jnp.full((16,), dd, jnp.int32)
                        val = plsc.load_gather(gref, [rows, col])
                        acc[dd] = acc[dd] + w * val
            for dd in range(_D):
                col = jnp.full((16,), dd, jnp.int32)
                plsc.store_scatter(out_v, [plocal, col], acc[dd])
            return 0

        lax.fori_loop(0, _B // 16, group_body, 0)
        pltpu.sync_copy(out_v, out_hbm.at[pl.ds(base, _B), :])
        return 0

    lax.fori_loop(0, nblk_w, block_body, 0)


def kernel(coord, hier_indices, features):
    offs = jnp.array([2 * _R, _R, 0], dtype=jnp.int32).reshape(3, 1, 1)
    hier_off = (hier_indices + offs).reshape(
        _LEVELS * _NBLK * _IDXROWS, _IDXW)
    feat_flat = features.reshape(_LEVELS * _R, _D)
    coord_flat = coord.T.reshape(3 * _N)

    octree = pl.kernel(
        _octree_body,
        out_type=jax.ShapeDtypeStruct((_N, _D), jnp.float32),
        mesh=plsc.VectorSubcoreMesh(core_axis_name="c", subcore_axis_name="s"),
        scratch_types=[
            pltpu.VMEM((_LEVELS, _IDXROWS, _IDXW), jnp.int32),
            pltpu.VMEM((_B * 8, _D), jnp.float32),
            pltpu.VMEM((_B * 8, _D), jnp.float32),
            pltpu.VMEM((_B * 8, _D), jnp.float32),
            pltpu.VMEM((_B,), jnp.float32),
            pltpu.VMEM((_B,), jnp.float32),
            pltpu.VMEM((_B,), jnp.float32),
            pltpu.VMEM((_B, _D), jnp.float32),
            pltpu.SemaphoreType.DMA,
        ],
        compiler_params=pltpu.CompilerParams(use_tc_tiling_on_sc=False,
                                             needs_layout_passes=False),
    )
    return octree(coord_flat, hier_off, feat_flat)
